# Initial kernel scaffold; baseline (speedup 1.0000x reference)
import jax
import jax.numpy as jnp
from jax.experimental import pallas as pl


def kernel(x, edge_index, edge_weight, W1, b1, W2, b2):
    raise NotImplementedError("write your pallas kernel here")


# speedup vs baseline: 14.8668x; 14.8668x over previous
import functools

import jax
import jax.numpy as jnp
from jax import lax
from jax.experimental import pallas as pl
from jax.experimental.pallas import tpu as pltpu
from jax.experimental.pallas import tpu_sc as plsc

NC = 2
NS = 16
NW = NC * NS
CH = 128
ROWS_PER_SUB = 640

_mesh = plsc.VectorSubcoreMesh(core_axis_name="c", subcore_axis_name="s")


def _zero_2d(ref, nrows, ncols):
    def body(r, _):
        for d in range(ncols // 16):
            ref[r, pl.ds(d * 16, 16)] = jnp.zeros((16,), ref.dtype)
        return 0
    lax.fori_loop(0, nrows, body, 0)


def _make_deg_kernel(nch, n_pad):
    @functools.partial(
        pl.kernel,
        out_type=jax.ShapeDtypeStruct((NC, n_pad), jnp.float32),
        mesh=_mesh,
        compiler_params=pltpu.CompilerParams(needs_layout_passes=False, use_tc_tiling_on_sc=False),
        scratch_types=[
            pltpu.VMEM((nch, CH), jnp.int32),
            pltpu.VMEM((nch, CH), jnp.float32),
            pltpu.VMEM((ROWS_PER_SUB,), jnp.float32),
            pltpu.VMEM_SHARED((n_pad,), jnp.float32),
        ],
    )
    def deg_kernel(col_hbm, ew_hbm, deg_out, col_v, ew_v, zbuf, deg_sh):
        cid = lax.axis_index("c")
        sid = lax.axis_index("s")
        wid = cid * NS + sid

        def zb(i, _):
            zbuf[pl.ds(i * 16, 16)] = jnp.zeros((16,), jnp.float32)
            return 0
        lax.fori_loop(0, ROWS_PER_SUB // 16, zb, 0)
        pltpu.sync_copy(zbuf, deg_sh.at[pl.ds(sid * ROWS_PER_SUB, ROWS_PER_SUB)])
        pltpu.sync_copy(col_hbm.at[wid], col_v)
        pltpu.sync_copy(ew_hbm.at[wid], ew_v)
        plsc.subcore_barrier()

        def body(ch, _):
            pltpu.sync_copy(ew_v.at[ch], deg_sh.at[col_v.at[ch]], add=True)
            return 0
        lax.fori_loop(0, nch, body, 0)
        plsc.subcore_barrier()
        pltpu.sync_copy(
            deg_sh.at[pl.ds(sid * ROWS_PER_SUB, ROWS_PER_SUB)],
            deg_out.at[cid, pl.ds(sid * ROWS_PER_SUB, ROWS_PER_SUB)],
        )

    return deg_kernel


def _make_agg_kernel(nch, n_pad, d):
    @functools.partial(
        pl.kernel,
        out_type=jax.ShapeDtypeStruct((NC, n_pad, d), jnp.float32),
        mesh=_mesh,
        compiler_params=pltpu.CompilerParams(needs_layout_passes=False, use_tc_tiling_on_sc=False),
        scratch_types=[
            pltpu.VMEM((nch, CH), jnp.int32),
            pltpu.VMEM((nch, CH), jnp.int32),
            pltpu.VMEM((nch * CH,), jnp.float32),
            pltpu.VMEM((CH, d), jnp.float32),
            pltpu.VMEM_SHARED((n_pad, d), jnp.float32),
            pltpu.SemaphoreType.DMA,
        ],
    )
    def agg_kernel(xs_hbm, row_hbm, col_hbm, ew_hbm, acc_out,
                   row_v, col_v, ew_v, g_v, acc_sh, sem):
        cid = lax.axis_index("c")
        sid = lax.axis_index("s")
        wid = cid * NS + sid

        _zero_2d(g_v, CH, d)
        for b in range(ROWS_PER_SUB // CH):
            pltpu.sync_copy(
                g_v, acc_sh.at[pl.ds(sid * ROWS_PER_SUB + b * CH, CH)])
        pltpu.sync_copy(row_hbm.at[wid], row_v)
        pltpu.sync_copy(col_hbm.at[wid], col_v)
        pltpu.sync_copy(ew_hbm.at[wid], ew_v)
        plsc.subcore_barrier()

        def body(ch, _):
            pltpu.async_copy(xs_hbm.at[row_v.at[ch]], g_v, sem).wait()

            def edge(j, _):
                ewb = plsc.load_gather(
                    ew_v, [jnp.full((16,), ch * CH + j, jnp.int32)])
                for dd in range(d // 16):
                    g_v[j, pl.ds(dd * 16, 16)] = g_v[j, pl.ds(dd * 16, 16)] * ewb
                return 0
            lax.fori_loop(0, CH, edge, 0, unroll=8)
            pltpu.sync_copy(g_v, acc_sh.at[col_v.at[ch]], add=True)
            return 0
        lax.fori_loop(0, nch, body, 0)
        plsc.subcore_barrier()
        pltpu.sync_copy(
            acc_sh.at[pl.ds(sid * ROWS_PER_SUB, ROWS_PER_SUB)],
            acc_out.at[cid, pl.ds(sid * ROWS_PER_SUB, ROWS_PER_SUB)],
        )

    return agg_kernel


def _dis_from_deg(degt_blk):
    deg = degt_blk[:, 0:1] + degt_blk[:, 1:2] + 1.0
    return jnp.where(deg > 0, lax.rsqrt(jnp.maximum(deg, 1e-12)), 0.0)


def _mm1_body(x_ref, w_ref, degt_ref, xs_ref):
    dis = _dis_from_deg(degt_ref[...])
    xw = jnp.dot(x_ref[...], w_ref[...], preferred_element_type=jnp.float32)
    xs_ref[...] = xw * dis


def _mm2_body(accp_ref, xs1_ref, degt_ref, b1_ref, w2_ref, xs2_ref):
    dis = _dis_from_deg(degt_ref[...])
    acc = accp_ref[0] + accp_ref[1] + xs1_ref[...]
    h = jnp.maximum(acc * dis + b1_ref[...], 0.0)
    xs2_ref[...] = jnp.dot(
        h, w2_ref[...], preferred_element_type=jnp.float32) * dis


def _final_body(accp_ref, xs2_ref, degt_ref, b2_ref, out_ref):
    dis = _dis_from_deg(degt_ref[...])
    acc = accp_ref[0] + accp_ref[1] + xs2_ref[...]
    out_ref[...] = jnp.maximum(acc * dis + b2_ref[...], 0.0)


def kernel(x, edge_index, edge_weight, W1, b1, W2, b2):
    n, d_in = x.shape
    d = W1.shape[1]
    e = edge_weight.shape[0]

    per_w = -(-e // NW)
    nch = -(-per_w // CH)
    e_pad = NW * nch * CH
    n_pad = NS * ROWS_PER_SUB

    row = edge_index[0].astype(jnp.int32)
    col = edge_index[1].astype(jnp.int32)
    pad = e_pad - e
    row3 = jnp.pad(row, (0, pad)).reshape(NW, nch, CH)
    col3 = jnp.pad(col, (0, pad)).reshape(NW, nch, CH)
    ew3 = jnp.pad(edge_weight.astype(jnp.float32), (0, pad)).reshape(NW, nch, CH)

    deg_part = _make_deg_kernel(nch, n_pad)(col3, ew3)
    degt = jnp.transpose(deg_part)[:n]

    R = 1000
    grid = (n // R,)
    degt_spec = pl.BlockSpec((R, 2), lambda i: (i, 0))
    nd_spec = pl.BlockSpec((R, d), lambda i: (i, 0))
    accp_spec = pl.BlockSpec((2, R, d), lambda i: (0, i, 0))
    b_spec = pl.BlockSpec((1, d), lambda i: (0, 0))

    xs1 = pl.pallas_call(
        _mm1_body,
        grid=grid,
        in_specs=[pl.BlockSpec((R, d_in), lambda i: (i, 0)),
                  pl.BlockSpec((d_in, d), lambda i: (0, 0)),
                  degt_spec],
        out_specs=nd_spec,
        out_shape=jax.ShapeDtypeStruct((n, d), jnp.float32),
    )(x, W1, degt)

    ew2 = ew3.reshape(NW, nch * CH)
    agg = _make_agg_kernel(nch, n_pad, d)
    acc1 = agg(xs1, row3, col3, ew2)

    xs2 = pl.pallas_call(
        _mm2_body,
        grid=grid,
        in_specs=[accp_spec, nd_spec, degt_spec, b_spec,
                  pl.BlockSpec((d, d), lambda i: (0, 0))],
        out_specs=nd_spec,
        out_shape=jax.ShapeDtypeStruct((n, d), jnp.float32),
    )(acc1[:, :n], xs1, degt, b1.reshape(1, d), W2)

    acc2 = agg(xs2, row3, col3, ew2)

    out = pl.pallas_call(
        _final_body,
        grid=grid,
        in_specs=[accp_spec, nd_spec, degt_spec, b_spec],
        out_specs=nd_spec,
        out_shape=jax.ShapeDtypeStruct((n, d), jnp.float32),
    )(acc2[:, :n], xs2, degt, b2.reshape(1, d))

    return out


# speedup vs baseline: 15.7803x; 1.0614x over previous
import functools

import jax
import jax.numpy as jnp
from jax import lax
from jax.experimental import pallas as pl
from jax.experimental.pallas import tpu as pltpu
from jax.experimental.pallas import tpu_sc as plsc

NC = 2
NS = 16
NW = NC * NS
CH = 128
ROWS_PER_SUB = 640

_mesh = plsc.VectorSubcoreMesh(core_axis_name="c", subcore_axis_name="s")


def _zero_2d(ref, nrows, ncols):
    def body(r, _):
        for d in range(ncols // 16):
            ref[r, pl.ds(d * 16, 16)] = jnp.zeros((16,), ref.dtype)
        return 0
    lax.fori_loop(0, nrows, body, 0)


def _make_deg_kernel(nch, n_pad):
    @functools.partial(
        pl.kernel,
        out_type=jax.ShapeDtypeStruct((NC, n_pad), jnp.float32),
        mesh=_mesh,
        compiler_params=pltpu.CompilerParams(needs_layout_passes=False, use_tc_tiling_on_sc=False),
        scratch_types=[
            pltpu.VMEM((nch, CH), jnp.int32),
            pltpu.VMEM((nch, CH), jnp.float32),
            pltpu.VMEM((ROWS_PER_SUB,), jnp.float32),
            pltpu.VMEM_SHARED((n_pad,), jnp.float32),
        ],
    )
    def deg_kernel(col_hbm, ew_hbm, deg_out, col_v, ew_v, zbuf, deg_sh):
        cid = lax.axis_index("c")
        sid = lax.axis_index("s")
        wid = cid * NS + sid

        def zb(i, _):
            zbuf[pl.ds(i * 16, 16)] = jnp.zeros((16,), jnp.float32)
            return 0
        lax.fori_loop(0, ROWS_PER_SUB // 16, zb, 0)
        pltpu.sync_copy(zbuf, deg_sh.at[pl.ds(sid * ROWS_PER_SUB, ROWS_PER_SUB)])
        pltpu.sync_copy(col_hbm.at[wid], col_v)
        pltpu.sync_copy(ew_hbm.at[wid], ew_v)
        plsc.subcore_barrier()

        def body(ch, _):
            pltpu.sync_copy(ew_v.at[ch], deg_sh.at[col_v.at[ch]], add=True)
            return 0
        lax.fori_loop(0, nch, body, 0)
        plsc.subcore_barrier()
        pltpu.sync_copy(
            deg_sh.at[pl.ds(sid * ROWS_PER_SUB, ROWS_PER_SUB)],
            deg_out.at[cid, pl.ds(sid * ROWS_PER_SUB, ROWS_PER_SUB)],
        )

    return deg_kernel


def _make_agg_kernel(nch, n_pad, d):
    assert nch % 4 == 0 and nch >= 8

    @functools.partial(
        pl.kernel,
        out_type=jax.ShapeDtypeStruct((NC, n_pad, d), jnp.float32),
        mesh=_mesh,
        compiler_params=pltpu.CompilerParams(needs_layout_passes=False, use_tc_tiling_on_sc=False),
        scratch_types=[
            pltpu.VMEM((nch, CH), jnp.int32),
            pltpu.VMEM((nch, CH), jnp.int32),
            pltpu.VMEM((nch * CH,), jnp.float32),
            pltpu.VMEM((CH, d), jnp.float32),
            pltpu.VMEM((CH, d), jnp.float32),
            pltpu.VMEM((CH, d), jnp.float32),
            pltpu.VMEM((CH, d), jnp.float32),
            pltpu.SemaphoreType.DMA,
            pltpu.SemaphoreType.DMA,
            pltpu.SemaphoreType.DMA,
            pltpu.SemaphoreType.DMA,
            pltpu.SemaphoreType.DMA,
            pltpu.SemaphoreType.DMA,
            pltpu.SemaphoreType.DMA,
            pltpu.SemaphoreType.DMA,
            pltpu.VMEM_SHARED((n_pad, d), jnp.float32),
        ],
    )
    def agg_kernel(xs_hbm, row_hbm, col_hbm, ew_hbm, acc_out,
                   row_v, col_v, ew_v, g0, g1, g2, g3,
                   gs0, gs1, gs2, gs3, ss0, ss1, ss2, ss3, acc_sh):
        cid = lax.axis_index("c")
        sid = lax.axis_index("s")
        wid = cid * NS + sid
        g = [g0, g1, g2, g3]
        gsem = [gs0, gs1, gs2, gs3]
        ssem = [ss0, ss1, ss2, ss3]

        _zero_2d(g0, CH, d)
        for b in range(ROWS_PER_SUB // CH):
            pltpu.sync_copy(
                g0, acc_sh.at[pl.ds(sid * ROWS_PER_SUB + b * CH, CH)])
        pltpu.sync_copy(row_hbm.at[wid], row_v)
        pltpu.sync_copy(col_hbm.at[wid], col_v)
        pltpu.sync_copy(ew_hbm.at[wid], ew_v)
        plsc.subcore_barrier()

        def scale(gb, ch):
            def edge(j, _):
                ewb = plsc.load_gather(
                    ew_v, [jnp.full((16,), ch * CH + j, jnp.int32)])
                for dd in range(d // 16):
                    gb[j, pl.ds(dd * 16, 16)] = gb[j, pl.ds(dd * 16, 16)] * ewb
                return 0
            lax.fori_loop(0, CH, edge, 0, unroll=8)

        pltpu.async_copy(xs_hbm.at[row_v.at[0]], g[0], gsem[0])
        pltpu.async_copy(xs_hbm.at[row_v.at[1]], g[1], gsem[1])

        def body(i, _):
            for b in range(4):
                ch = 4 * i + b
                b2 = (b + 2) % 4
                pltpu.make_async_copy(
                    xs_hbm.at[row_v.at[ch]], g[b], gsem[b]).wait()
                scale(g[b], ch)
                pltpu.async_copy(g[b], acc_sh.at[col_v.at[ch]], ssem[b],
                                 add=True)

                @pl.when(ch >= 2)
                def _():
                    pltpu.make_async_copy(
                        g[b2], acc_sh.at[col_v.at[ch - 2]], ssem[b2]).wait()

                @pl.when(ch + 2 < nch)
                def _():
                    pltpu.async_copy(
                        xs_hbm.at[row_v.at[ch + 2]], g[b2], gsem[b2])
            return 0
        lax.fori_loop(0, nch // 4, body, 0)

        for ch in (nch - 2, nch - 1):
            pltpu.make_async_copy(
                g[ch % 4], acc_sh.at[col_v.at[ch]], ssem[ch % 4]).wait()
        plsc.subcore_barrier()
        pltpu.sync_copy(
            acc_sh.at[pl.ds(sid * ROWS_PER_SUB, ROWS_PER_SUB)],
            acc_out.at[cid, pl.ds(sid * ROWS_PER_SUB, ROWS_PER_SUB)],
        )

    return agg_kernel


def _dis_from_deg(degt_blk):
    deg = degt_blk[:, 0:1] + degt_blk[:, 1:2] + 1.0
    return jnp.where(deg > 0, lax.rsqrt(jnp.maximum(deg, 1e-12)), 0.0)


def _mm1_body(x_ref, w_ref, degt_ref, xs_ref):
    dis = _dis_from_deg(degt_ref[...])
    xw = jnp.dot(x_ref[...], w_ref[...], preferred_element_type=jnp.float32)
    xs_ref[...] = xw * dis


def _mm2_body(accp_ref, xs1_ref, degt_ref, b1_ref, w2_ref, xs2_ref):
    dis = _dis_from_deg(degt_ref[...])
    acc = accp_ref[0] + accp_ref[1] + xs1_ref[...]
    h = jnp.maximum(acc * dis + b1_ref[...], 0.0)
    xs2_ref[...] = jnp.dot(
        h, w2_ref[...], preferred_element_type=jnp.float32) * dis


def _final_body(accp_ref, xs2_ref, degt_ref, b2_ref, out_ref):
    dis = _dis_from_deg(degt_ref[...])
    acc = accp_ref[0] + accp_ref[1] + xs2_ref[...]
    out_ref[...] = jnp.maximum(acc * dis + b2_ref[...], 0.0)


def kernel(x, edge_index, edge_weight, W1, b1, W2, b2):
    n, d_in = x.shape
    d = W1.shape[1]
    e = edge_weight.shape[0]

    per_w = -(-e // NW)
    nch = -(-per_w // CH)
    nch = -(-nch // 4) * 4
    e_pad = NW * nch * CH
    n_pad = NS * ROWS_PER_SUB

    row = edge_index[0].astype(jnp.int32)
    col = edge_index[1].astype(jnp.int32)
    pad = e_pad - e
    row3 = jnp.pad(row, (0, pad)).reshape(NW, nch, CH)
    col3 = jnp.pad(col, (0, pad)).reshape(NW, nch, CH)
    ew3 = jnp.pad(edge_weight.astype(jnp.float32), (0, pad)).reshape(NW, nch, CH)

    deg_part = _make_deg_kernel(nch, n_pad)(col3, ew3)
    degt = jnp.transpose(deg_part)[:n]

    R = 1000
    grid = (n // R,)
    degt_spec = pl.BlockSpec((R, 2), lambda i: (i, 0))
    nd_spec = pl.BlockSpec((R, d), lambda i: (i, 0))
    accp_spec = pl.BlockSpec((2, R, d), lambda i: (0, i, 0))
    b_spec = pl.BlockSpec((1, d), lambda i: (0, 0))

    xs1 = pl.pallas_call(
        _mm1_body,
        grid=grid,
        in_specs=[pl.BlockSpec((R, d_in), lambda i: (i, 0)),
                  pl.BlockSpec((d_in, d), lambda i: (0, 0)),
                  degt_spec],
        out_specs=nd_spec,
        out_shape=jax.ShapeDtypeStruct((n, d), jnp.float32),
    )(x, W1, degt)

    ew2 = ew3.reshape(NW, nch * CH)
    agg = _make_agg_kernel(nch, n_pad, d)
    acc1 = agg(xs1, row3, col3, ew2)

    xs2 = pl.pallas_call(
        _mm2_body,
        grid=grid,
        in_specs=[accp_spec, nd_spec, degt_spec, b_spec,
                  pl.BlockSpec((d, d), lambda i: (0, 0))],
        out_specs=nd_spec,
        out_shape=jax.ShapeDtypeStruct((n, d), jnp.float32),
    )(acc1[:, :n], xs1, degt, b1.reshape(1, d), W2)

    acc2 = agg(xs2, row3, col3, ew2)

    out = pl.pallas_call(
        _final_body,
        grid=grid,
        in_specs=[accp_spec, nd_spec, degt_spec, b_spec],
        out_specs=nd_spec,
        out_shape=jax.ShapeDtypeStruct((n, d), jnp.float32),
    )(acc2[:, :n], xs2, degt, b2.reshape(1, d))

    return out


# speedup vs baseline: 24.2476x; 1.5366x over previous
import functools

import jax
import jax.numpy as jnp
from jax import lax
from jax.experimental import pallas as pl
from jax.experimental.pallas import tpu as pltpu
from jax.experimental.pallas import tpu_sc as plsc

NC = 2
NS = 16
NW = NC * NS
CH = 128
ROWS_PER_SUB = 640

_mesh = plsc.VectorSubcoreMesh(core_axis_name="c", subcore_axis_name="s")


def _zero_2d(ref, nrows, ncols):
    def body(r, _):
        for d in range(ncols // 16):
            ref[r, pl.ds(d * 16, 16)] = jnp.zeros((16,), ref.dtype)
        return 0
    lax.fori_loop(0, nrows, body, 0)


def _make_deg_kernel(nch, n_pad):
    @functools.partial(
        pl.kernel,
        out_type=jax.ShapeDtypeStruct((NC, n_pad), jnp.float32),
        mesh=_mesh,
        compiler_params=pltpu.CompilerParams(needs_layout_passes=False, use_tc_tiling_on_sc=False),
        scratch_types=[
            pltpu.VMEM((nch, CH), jnp.int32),
            pltpu.VMEM((nch, CH), jnp.float32),
            pltpu.VMEM((ROWS_PER_SUB,), jnp.float32),
            pltpu.VMEM_SHARED((n_pad,), jnp.float32),
        ],
    )
    def deg_kernel(col_hbm, ew_hbm, deg_out, col_v, ew_v, zbuf, deg_sh):
        cid = lax.axis_index("c")
        sid = lax.axis_index("s")
        wid = cid * NS + sid

        def zb(i, _):
            zbuf[pl.ds(i * 16, 16)] = jnp.zeros((16,), jnp.float32)
            return 0
        lax.fori_loop(0, ROWS_PER_SUB // 16, zb, 0)
        pltpu.sync_copy(zbuf, deg_sh.at[pl.ds(sid * ROWS_PER_SUB, ROWS_PER_SUB)])
        pltpu.sync_copy(col_hbm.at[wid], col_v)
        pltpu.sync_copy(ew_hbm.at[wid], ew_v)
        plsc.subcore_barrier()

        def body(ch, _):
            pltpu.sync_copy(ew_v.at[ch], deg_sh.at[col_v.at[ch]], add=True)
            return 0
        lax.fori_loop(0, nch, body, 0)
        plsc.subcore_barrier()
        pltpu.sync_copy(
            deg_sh.at[pl.ds(sid * ROWS_PER_SUB, ROWS_PER_SUB)],
            deg_out.at[cid, pl.ds(sid * ROWS_PER_SUB, ROWS_PER_SUB)],
        )

    return deg_kernel


def _make_agg_kernel(nch, n_pad, dh):
    assert nch % 4 == 0 and nch >= 8

    @functools.partial(
        pl.kernel,
        out_type=jax.ShapeDtypeStruct((NC, n_pad, dh), jnp.float32),
        mesh=_mesh,
        compiler_params=pltpu.CompilerParams(needs_layout_passes=False, use_tc_tiling_on_sc=False),
        scratch_types=[
            pltpu.VMEM((nch, CH), jnp.int32),
            pltpu.VMEM((nch, CH), jnp.int32),
            pltpu.VMEM((nch * CH,), jnp.float32),
            pltpu.VMEM((CH, dh), jnp.float32),
            pltpu.VMEM((CH, dh), jnp.float32),
            pltpu.VMEM((CH, dh), jnp.float32),
            pltpu.VMEM((CH, dh), jnp.float32),
            pltpu.SemaphoreType.DMA,
            pltpu.SemaphoreType.DMA,
            pltpu.SemaphoreType.DMA,
            pltpu.SemaphoreType.DMA,
            pltpu.SemaphoreType.DMA,
            pltpu.SemaphoreType.DMA,
            pltpu.SemaphoreType.DMA,
            pltpu.SemaphoreType.DMA,
            pltpu.VMEM_SHARED((n_pad, dh), jnp.float32),
            pltpu.VMEM_SHARED((n_pad, dh), jnp.float32),
        ],
    )
    def agg_kernel(xs_hbm, row_hbm, col_hbm, ew_hbm, acc_out,
                   row_v, col_v, ew_v, g0, g1, g2, g3,
                   gs0, gs1, gs2, gs3, ss0, ss1, ss2, ss3, acc_sh, xs_sh):
        cid = lax.axis_index("c")
        sid = lax.axis_index("s")
        g = [g0, g1, g2, g3]
        gsem = [gs0, gs1, gs2, gs3]
        ssem = [ss0, ss1, ss2, ss3]

        _zero_2d(g0, CH, dh)
        for b in range(ROWS_PER_SUB // CH):
            pltpu.sync_copy(
                g0, acc_sh.at[pl.ds(sid * ROWS_PER_SUB + b * CH, CH)])
        pltpu.sync_copy(row_hbm.at[sid], row_v)
        pltpu.sync_copy(col_hbm.at[sid], col_v)
        pltpu.sync_copy(ew_hbm.at[sid], ew_v)
        n_rows = xs_hbm.shape[1]
        stripe = n_rows // NS
        pltpu.sync_copy(xs_hbm.at[cid].at[pl.ds(sid * stripe, stripe)],
                        xs_sh.at[pl.ds(sid * stripe, stripe)])
        plsc.subcore_barrier()

        def scale(gb, ch):
            def edge(j, _):
                ewb = plsc.load_gather(
                    ew_v, [jnp.full((16,), ch * CH + j, jnp.int32)])
                for dd in range(dh // 16):
                    gb[j, pl.ds(dd * 16, 16)] = gb[j, pl.ds(dd * 16, 16)] * ewb
                return 0
            lax.fori_loop(0, CH, edge, 0, unroll=8)

        pltpu.async_copy(xs_sh.at[row_v.at[0]], g[0], gsem[0])
        pltpu.async_copy(xs_sh.at[row_v.at[1]], g[1], gsem[1])

        def body(i, _):
            for b in range(4):
                ch = 4 * i + b
                b2 = (b + 2) % 4
                pltpu.make_async_copy(
                    xs_sh.at[row_v.at[ch]], g[b], gsem[b]).wait()
                scale(g[b], ch)
                pltpu.async_copy(g[b], acc_sh.at[col_v.at[ch]], ssem[b],
                                 add=True)

                @pl.when(ch >= 2)
                def _():
                    pltpu.make_async_copy(
                        g[b2], acc_sh.at[col_v.at[ch - 2]], ssem[b2]).wait()

                @pl.when(ch + 2 < nch)
                def _():
                    pltpu.async_copy(
                        xs_sh.at[row_v.at[ch + 2]], g[b2], gsem[b2])
            return 0
        lax.fori_loop(0, nch // 4, body, 0)

        for ch in (nch - 2, nch - 1):
            pltpu.make_async_copy(
                g[ch % 4], acc_sh.at[col_v.at[ch]], ssem[ch % 4]).wait()
        plsc.subcore_barrier()
        pltpu.sync_copy(
            acc_sh.at[pl.ds(sid * ROWS_PER_SUB, ROWS_PER_SUB)],
            acc_out.at[cid, pl.ds(sid * ROWS_PER_SUB, ROWS_PER_SUB)],
        )

    return agg_kernel


def _dis_from_deg(degt_blk):
    deg = degt_blk[:, 0:1] + degt_blk[:, 1:2] + 1.0
    return jnp.where(deg > 0, lax.rsqrt(jnp.maximum(deg, 1e-12)), 0.0)


def _mm1_body(x_ref, w_ref, degt_ref, xs_ref):
    dis = _dis_from_deg(degt_ref[...])
    xw = jnp.dot(x_ref[...], w_ref[...], preferred_element_type=jnp.float32)
    xw = xw * dis
    dh = xw.shape[1] // 2
    xs_ref[0] = xw[:, :dh]
    xs_ref[1] = xw[:, dh:]


def _mm2_body(accp_ref, xsp_ref, degt_ref, b1_ref, w2_ref, xs2_ref):
    dis = _dis_from_deg(degt_ref[...])
    acc = jnp.concatenate(
        [accp_ref[0] + xsp_ref[0], accp_ref[1] + xsp_ref[1]], axis=-1)
    h = jnp.maximum(acc * dis + b1_ref[...], 0.0)
    xw2 = jnp.dot(h, w2_ref[...], preferred_element_type=jnp.float32) * dis
    dh = xw2.shape[1] // 2
    xs2_ref[0] = xw2[:, :dh]
    xs2_ref[1] = xw2[:, dh:]


def _final_body(accp_ref, xsp_ref, degt_ref, b2_ref, out_ref):
    dis = _dis_from_deg(degt_ref[...])
    acc = jnp.concatenate(
        [accp_ref[0] + xsp_ref[0], accp_ref[1] + xsp_ref[1]], axis=-1)
    out_ref[...] = jnp.maximum(acc * dis + b2_ref[...], 0.0)


def kernel(x, edge_index, edge_weight, W1, b1, W2, b2):
    n, d_in = x.shape
    d = W1.shape[1]
    dh = d // 2
    e = edge_weight.shape[0]

    per_w = -(-e // NW)
    nch_d = -(-per_w // CH)
    e_pad_d = NW * nch_d * CH
    per_t = -(-e // NS)
    nch_a = -(-(-(-per_t // CH)) // 4) * 4
    e_pad_a = NS * nch_a * CH
    n_pad = NS * ROWS_PER_SUB

    row = edge_index[0].astype(jnp.int32)
    col = edge_index[1].astype(jnp.int32)
    ew = edge_weight.astype(jnp.float32)
    col_d = jnp.pad(col, (0, e_pad_d - e)).reshape(NW, nch_d, CH)
    ew_d = jnp.pad(ew, (0, e_pad_d - e)).reshape(NW, nch_d, CH)
    row_a = jnp.pad(row, (0, e_pad_a - e)).reshape(NS, nch_a, CH)
    col_a = jnp.pad(col, (0, e_pad_a - e)).reshape(NS, nch_a, CH)
    ew_a = jnp.pad(ew, (0, e_pad_a - e)).reshape(NS, nch_a * CH)

    deg_part = _make_deg_kernel(nch_d, n_pad)(col_d, ew_d)
    degt = jnp.transpose(deg_part)[:n]

    R = 1000
    grid = (n // R,)
    degt_spec = pl.BlockSpec((R, 2), lambda i: (i, 0))
    half_spec = pl.BlockSpec((2, R, dh), lambda i: (0, i, 0))
    b_spec = pl.BlockSpec((1, d), lambda i: (0, 0))
    half_shape = jax.ShapeDtypeStruct((2, n, dh), jnp.float32)

    xs1 = pl.pallas_call(
        _mm1_body,
        grid=grid,
        in_specs=[pl.BlockSpec((R, d_in), lambda i: (i, 0)),
                  pl.BlockSpec((d_in, d), lambda i: (0, 0)),
                  degt_spec],
        out_specs=half_spec,
        out_shape=half_shape,
    )(x, W1, degt)

    agg = _make_agg_kernel(nch_a, n_pad, dh)
    acc1 = agg(xs1, row_a, col_a, ew_a)

    xs2 = pl.pallas_call(
        _mm2_body,
        grid=grid,
        in_specs=[half_spec, half_spec, degt_spec, b_spec,
                  pl.BlockSpec((d, d), lambda i: (0, 0))],
        out_specs=half_spec,
        out_shape=half_shape,
    )(acc1[:, :n], xs1, degt, b1.reshape(1, d), W2)

    acc2 = agg(xs2, row_a, col_a, ew_a)

    out = pl.pallas_call(
        _final_body,
        grid=grid,
        in_specs=[half_spec, half_spec, degt_spec, b_spec],
        out_specs=pl.BlockSpec((R, d), lambda i: (i, 0)),
        out_shape=jax.ShapeDtypeStruct((n, d), jnp.float32),
    )(acc2[:, :n], xs2, degt, b2.reshape(1, d))

    return out


# speedup vs baseline: 31.7051x; 1.3076x over previous
import functools

import jax
import jax.numpy as jnp
from jax import lax
from jax.experimental import pallas as pl
from jax.experimental.pallas import tpu as pltpu
from jax.experimental.pallas import tpu_sc as plsc

NC = 2
NS = 16
NW = NC * NS
CH = 128
ROWS_PER_SUB = 640

_mesh = plsc.VectorSubcoreMesh(core_axis_name="c", subcore_axis_name="s")


def _zero_2d(ref, nrows, ncols):
    def body(r, _):
        for d in range(ncols // 16):
            ref[r, pl.ds(d * 16, 16)] = jnp.zeros((16,), ref.dtype)
        return 0
    lax.fori_loop(0, nrows, body, 0)


def _make_deg_kernel(nch, n_pad):
    @functools.partial(
        pl.kernel,
        out_type=jax.ShapeDtypeStruct((NC, n_pad), jnp.float32),
        mesh=_mesh,
        compiler_params=pltpu.CompilerParams(needs_layout_passes=False, use_tc_tiling_on_sc=False),
        scratch_types=[
            pltpu.VMEM((nch, CH), jnp.int32),
            pltpu.VMEM((nch, CH), jnp.float32),
            pltpu.VMEM((ROWS_PER_SUB,), jnp.float32),
            pltpu.VMEM_SHARED((n_pad,), jnp.float32),
        ],
    )
    def deg_kernel(col_hbm, ew_hbm, deg_out, col_v, ew_v, zbuf, deg_sh):
        cid = lax.axis_index("c")
        sid = lax.axis_index("s")
        wid = cid * NS + sid

        def zb(i, _):
            zbuf[pl.ds(i * 16, 16)] = jnp.zeros((16,), jnp.float32)
            return 0
        lax.fori_loop(0, ROWS_PER_SUB // 16, zb, 0)
        pltpu.sync_copy(zbuf, deg_sh.at[pl.ds(sid * ROWS_PER_SUB, ROWS_PER_SUB)])
        pltpu.sync_copy(col_hbm.at[wid], col_v)
        pltpu.sync_copy(ew_hbm.at[wid], ew_v)
        plsc.subcore_barrier()

        def body(ch, _):
            pltpu.sync_copy(ew_v.at[ch], deg_sh.at[col_v.at[ch]], add=True)
            return 0
        lax.fori_loop(0, nch, body, 0)
        plsc.subcore_barrier()
        pltpu.sync_copy(
            deg_sh.at[pl.ds(sid * ROWS_PER_SUB, ROWS_PER_SUB)],
            deg_out.at[cid, pl.ds(sid * ROWS_PER_SUB, ROWS_PER_SUB)],
        )

    return deg_kernel


def _make_agg_kernel(nch, n_pad, dh):
    assert nch % 4 == 0 and nch >= 8

    @functools.partial(
        pl.kernel,
        out_type=jax.ShapeDtypeStruct((NC, n_pad, dh), jnp.float32),
        mesh=_mesh,
        compiler_params=pltpu.CompilerParams(needs_layout_passes=False, use_tc_tiling_on_sc=False),
        scratch_types=[
            pltpu.VMEM((nch, CH), jnp.int32),
            pltpu.VMEM((nch, CH), jnp.int32),
            pltpu.VMEM((nch * CH,), jnp.float32),
            pltpu.VMEM((CH, dh), jnp.float32),
            pltpu.VMEM((CH, dh), jnp.float32),
            pltpu.VMEM((CH, dh), jnp.float32),
            pltpu.VMEM((CH, dh), jnp.float32),
            pltpu.SemaphoreType.DMA,
            pltpu.SemaphoreType.DMA,
            pltpu.SemaphoreType.DMA,
            pltpu.SemaphoreType.DMA,
            pltpu.SemaphoreType.DMA,
            pltpu.SemaphoreType.DMA,
            pltpu.SemaphoreType.DMA,
            pltpu.SemaphoreType.DMA,
            pltpu.VMEM_SHARED((n_pad, dh), jnp.float32),
            pltpu.VMEM_SHARED((n_pad, dh), jnp.float32),
        ],
    )
    def agg_kernel(xs_hbm, row_hbm, col_hbm, ew_hbm, acc_out,
                   row_v, col_v, ew_v, g0, g1, g2, g3,
                   gs0, gs1, gs2, gs3, ss0, ss1, ss2, ss3, acc_sh, xs_sh):
        cid = lax.axis_index("c")
        sid = lax.axis_index("s")
        g = [g0, g1, g2, g3]
        gsem = [gs0, gs1, gs2, gs3]
        ssem = [ss0, ss1, ss2, ss3]

        _zero_2d(g0, CH, dh)
        for b in range(ROWS_PER_SUB // CH):
            pltpu.sync_copy(
                g0, acc_sh.at[pl.ds(sid * ROWS_PER_SUB + b * CH, CH)])
        pltpu.sync_copy(row_hbm.at[sid], row_v)
        pltpu.sync_copy(col_hbm.at[sid], col_v)
        pltpu.sync_copy(ew_hbm.at[sid], ew_v)
        n_rows = xs_hbm.shape[1]
        stripe = n_rows // NS
        pltpu.sync_copy(xs_hbm.at[cid].at[pl.ds(sid * stripe, stripe)],
                        xs_sh.at[pl.ds(sid * stripe, stripe)])
        plsc.subcore_barrier()

        def scale(gb, ch):
            def grp(k, _):
                ew16 = ew_v[pl.ds(ch * CH + k * 16, 16)]
                for l in range(16):
                    j = k * 16 + l
                    ewb = jnp.broadcast_to(ew16[l], (16,))
                    for dd in range(dh // 16):
                        gb[j, pl.ds(dd * 16, 16)] = (
                            gb[j, pl.ds(dd * 16, 16)] * ewb)
                return 0
            lax.fori_loop(0, CH // 16, grp, 0)

        pltpu.async_copy(xs_sh.at[row_v.at[0]], g[0], gsem[0])
        pltpu.async_copy(xs_sh.at[row_v.at[1]], g[1], gsem[1])

        def body(i, _):
            for b in range(4):
                ch = 4 * i + b
                b2 = (b + 2) % 4
                pltpu.make_async_copy(
                    xs_sh.at[row_v.at[ch]], g[b], gsem[b]).wait()
                scale(g[b], ch)
                pltpu.async_copy(g[b], acc_sh.at[col_v.at[ch]], ssem[b],
                                 add=True)

                @pl.when(ch >= 2)
                def _():
                    pltpu.make_async_copy(
                        g[b2], acc_sh.at[col_v.at[ch - 2]], ssem[b2]).wait()

                @pl.when(ch + 2 < nch)
                def _():
                    pltpu.async_copy(
                        xs_sh.at[row_v.at[ch + 2]], g[b2], gsem[b2])
            return 0
        lax.fori_loop(0, nch // 4, body, 0)

        for ch in (nch - 2, nch - 1):
            pltpu.make_async_copy(
                g[ch % 4], acc_sh.at[col_v.at[ch]], ssem[ch % 4]).wait()
        plsc.subcore_barrier()
        pltpu.sync_copy(
            acc_sh.at[pl.ds(sid * ROWS_PER_SUB, ROWS_PER_SUB)],
            acc_out.at[cid, pl.ds(sid * ROWS_PER_SUB, ROWS_PER_SUB)],
        )

    return agg_kernel


def _dis_from_deg(degt_blk):
    deg = degt_blk[:, 0:1] + degt_blk[:, 1:2] + 1.0
    return jnp.where(deg > 0, lax.rsqrt(jnp.maximum(deg, 1e-12)), 0.0)


def _mm1_body(x_ref, w_ref, degt_ref, xs_ref):
    dis = _dis_from_deg(degt_ref[...])
    xw = jnp.dot(x_ref[...], w_ref[...], preferred_element_type=jnp.float32)
    xw = xw * dis
    dh = xw.shape[1] // 2
    xs_ref[0] = xw[:, :dh]
    xs_ref[1] = xw[:, dh:]


def _mm2_body(accp_ref, xsp_ref, degt_ref, b1_ref, w2_ref, xs2_ref):
    dis = _dis_from_deg(degt_ref[...])
    acc = jnp.concatenate(
        [accp_ref[0] + xsp_ref[0], accp_ref[1] + xsp_ref[1]], axis=-1)
    h = jnp.maximum(acc * dis + b1_ref[...], 0.0)
    xw2 = jnp.dot(h, w2_ref[...], preferred_element_type=jnp.float32) * dis
    dh = xw2.shape[1] // 2
    xs2_ref[0] = xw2[:, :dh]
    xs2_ref[1] = xw2[:, dh:]


def _final_body(accp_ref, xsp_ref, degt_ref, b2_ref, out_ref):
    dis = _dis_from_deg(degt_ref[...])
    acc = jnp.concatenate(
        [accp_ref[0] + xsp_ref[0], accp_ref[1] + xsp_ref[1]], axis=-1)
    out_ref[...] = jnp.maximum(acc * dis + b2_ref[...], 0.0)


def kernel(x, edge_index, edge_weight, W1, b1, W2, b2):
    n, d_in = x.shape
    d = W1.shape[1]
    dh = d // 2
    e = edge_weight.shape[0]

    per_w = -(-e // NW)
    nch_d = -(-per_w // CH)
    e_pad_d = NW * nch_d * CH
    per_t = -(-e // NS)
    nch_a = -(-(-(-per_t // CH)) // 4) * 4
    e_pad_a = NS * nch_a * CH
    n_pad = NS * ROWS_PER_SUB

    row = edge_index[0].astype(jnp.int32)
    col = edge_index[1].astype(jnp.int32)
    ew = edge_weight.astype(jnp.float32)
    col_d = jnp.pad(col, (0, e_pad_d - e)).reshape(NW, nch_d, CH)
    ew_d = jnp.pad(ew, (0, e_pad_d - e)).reshape(NW, nch_d, CH)
    row_a = jnp.pad(row, (0, e_pad_a - e)).reshape(NS, nch_a, CH)
    col_a = jnp.pad(col, (0, e_pad_a - e)).reshape(NS, nch_a, CH)
    ew_a = jnp.pad(ew, (0, e_pad_a - e)).reshape(NS, nch_a * CH)

    deg_part = _make_deg_kernel(nch_d, n_pad)(col_d, ew_d)
    degt = jnp.transpose(deg_part)[:n]

    R = 1000
    grid = (n // R,)
    degt_spec = pl.BlockSpec((R, 2), lambda i: (i, 0))
    half_spec = pl.BlockSpec((2, R, dh), lambda i: (0, i, 0))
    b_spec = pl.BlockSpec((1, d), lambda i: (0, 0))
    half_shape = jax.ShapeDtypeStruct((2, n, dh), jnp.float32)

    xs1 = pl.pallas_call(
        _mm1_body,
        grid=grid,
        in_specs=[pl.BlockSpec((R, d_in), lambda i: (i, 0)),
                  pl.BlockSpec((d_in, d), lambda i: (0, 0)),
                  degt_spec],
        out_specs=half_spec,
        out_shape=half_shape,
    )(x, W1, degt)

    agg = _make_agg_kernel(nch_a, n_pad, dh)
    acc1 = agg(xs1, row_a, col_a, ew_a)

    xs2 = pl.pallas_call(
        _mm2_body,
        grid=grid,
        in_specs=[half_spec, half_spec, degt_spec, b_spec,
                  pl.BlockSpec((d, d), lambda i: (0, 0))],
        out_specs=half_spec,
        out_shape=half_shape,
    )(acc1[:, :n], xs1, degt, b1.reshape(1, d), W2)

    acc2 = agg(xs2, row_a, col_a, ew_a)

    out = pl.pallas_call(
        _final_body,
        grid=grid,
        in_specs=[half_spec, half_spec, degt_spec, b_spec],
        out_specs=pl.BlockSpec((R, d), lambda i: (i, 0)),
        out_shape=jax.ShapeDtypeStruct((n, d), jnp.float32),
    )(acc2[:, :n], xs2, degt, b2.reshape(1, d))

    return out


# speedup vs baseline: 33.3827x; 1.0529x over previous
import functools

import jax
import jax.numpy as jnp
from jax import lax
from jax.experimental import pallas as pl
from jax.experimental.pallas import tpu as pltpu
from jax.experimental.pallas import tpu_sc as plsc

NC = 2
NS = 16
NW = NC * NS
CH = 128
ROWS_PER_SUB = 640

_mesh = plsc.VectorSubcoreMesh(core_axis_name="c", subcore_axis_name="s")


def _zero_2d(ref, nrows, ncols):
    def body(r, _):
        for d in range(ncols // 16):
            ref[r, pl.ds(d * 16, 16)] = jnp.zeros((16,), ref.dtype)
        return 0
    lax.fori_loop(0, nrows, body, 0)


def _make_deg_kernel(nch, n_pad):
    @functools.partial(
        pl.kernel,
        out_type=jax.ShapeDtypeStruct((NC, n_pad), jnp.float32),
        mesh=_mesh,
        compiler_params=pltpu.CompilerParams(needs_layout_passes=False, use_tc_tiling_on_sc=False),
        scratch_types=[
            pltpu.VMEM((nch, CH), jnp.int32),
            pltpu.VMEM((nch, CH), jnp.float32),
            pltpu.VMEM((ROWS_PER_SUB,), jnp.float32),
            pltpu.VMEM_SHARED((n_pad,), jnp.float32),
        ],
    )
    def deg_kernel(col_hbm, ew_hbm, deg_out, col_v, ew_v, zbuf, deg_sh):
        cid = lax.axis_index("c")
        sid = lax.axis_index("s")
        wid = cid * NS + sid

        def zb(i, _):
            zbuf[pl.ds(i * 16, 16)] = jnp.zeros((16,), jnp.float32)
            return 0
        lax.fori_loop(0, ROWS_PER_SUB // 16, zb, 0)
        pltpu.sync_copy(zbuf, deg_sh.at[pl.ds(sid * ROWS_PER_SUB, ROWS_PER_SUB)])
        pltpu.sync_copy(col_hbm.at[wid], col_v)
        pltpu.sync_copy(ew_hbm.at[wid], ew_v)
        plsc.subcore_barrier()

        def body(ch, _):
            pltpu.sync_copy(ew_v.at[ch], deg_sh.at[col_v.at[ch]], add=True)
            return 0
        lax.fori_loop(0, nch, body, 0)
        plsc.subcore_barrier()
        pltpu.sync_copy(
            deg_sh.at[pl.ds(sid * ROWS_PER_SUB, ROWS_PER_SUB)],
            deg_out.at[cid, pl.ds(sid * ROWS_PER_SUB, ROWS_PER_SUB)],
        )

    return deg_kernel


def _make_agg_kernel(nch, n_pad, dh):
    assert nch % 4 == 0 and nch >= 8

    @functools.partial(
        pl.kernel,
        out_type=jax.ShapeDtypeStruct((NC, n_pad, dh), jnp.float32),
        mesh=_mesh,
        compiler_params=pltpu.CompilerParams(needs_layout_passes=False, use_tc_tiling_on_sc=False),
        scratch_types=[
            pltpu.VMEM((nch, CH), jnp.int32),
            pltpu.VMEM((nch, CH), jnp.int32),
            pltpu.VMEM((nch * CH,), jnp.float32),
            pltpu.VMEM((CH, dh), jnp.float32),
            pltpu.VMEM((CH, dh), jnp.float32),
            pltpu.VMEM((CH, dh), jnp.float32),
            pltpu.VMEM((CH, dh), jnp.float32),
            pltpu.SemaphoreType.DMA,
            pltpu.SemaphoreType.DMA,
            pltpu.SemaphoreType.DMA,
            pltpu.SemaphoreType.DMA,
            pltpu.SemaphoreType.DMA,
            pltpu.SemaphoreType.DMA,
            pltpu.SemaphoreType.DMA,
            pltpu.SemaphoreType.DMA,
            pltpu.VMEM_SHARED((n_pad, dh), jnp.float32),
            pltpu.VMEM_SHARED((n_pad, dh), jnp.float32),
        ],
    )
    def agg_kernel(xs_hbm, row_hbm, col_hbm, ew_hbm, acc_out,
                   row_v, col_v, ew_v, g0, g1, g2, g3,
                   gs0, gs1, gs2, gs3, ss0, ss1, ss2, ss3, acc_sh, xs_sh):
        cid = lax.axis_index("c")
        sid = lax.axis_index("s")
        g = [g0, g1, g2, g3]
        gsem = [gs0, gs1, gs2, gs3]
        ssem = [ss0, ss1, ss2, ss3]

        _zero_2d(g0, CH, dh)
        for b in range(ROWS_PER_SUB // CH):
            pltpu.sync_copy(
                g0, acc_sh.at[pl.ds(sid * ROWS_PER_SUB + b * CH, CH)])
        pltpu.sync_copy(row_hbm.at[sid], row_v)
        pltpu.sync_copy(col_hbm.at[sid], col_v)
        pltpu.sync_copy(ew_hbm.at[sid], ew_v)
        n_rows = xs_hbm.shape[1]
        stripe = n_rows // NS
        pltpu.sync_copy(xs_hbm.at[cid].at[pl.ds(sid * stripe, stripe)],
                        xs_sh.at[pl.ds(sid * stripe, stripe)])
        plsc.subcore_barrier()

        def scale(gb, ch):
            def grp(k, _):
                ew16 = ew_v[pl.ds(ch * CH + k * 16, 16)]
                for l in range(16):
                    j = k * 16 + l
                    ewb = jnp.broadcast_to(ew16[l], (16,))
                    for dd in range(dh // 16):
                        gb[j, pl.ds(dd * 16, 16)] = (
                            gb[j, pl.ds(dd * 16, 16)] * ewb)
                return 0
            lax.fori_loop(0, CH // 16, grp, 0, unroll=4)

        pltpu.async_copy(xs_sh.at[row_v.at[0]], g[0], gsem[0])
        pltpu.async_copy(xs_sh.at[row_v.at[1]], g[1], gsem[1])

        def body(i, _):
            for b in range(4):
                ch = 4 * i + b
                b2 = (b + 2) % 4
                pltpu.make_async_copy(
                    xs_sh.at[row_v.at[ch]], g[b], gsem[b]).wait()
                scale(g[b], ch)
                pltpu.async_copy(g[b], acc_sh.at[col_v.at[ch]], ssem[b],
                                 add=True)

                @pl.when(ch >= 2)
                def _():
                    pltpu.make_async_copy(
                        g[b2], acc_sh.at[col_v.at[ch - 2]], ssem[b2]).wait()

                @pl.when(ch + 2 < nch)
                def _():
                    pltpu.async_copy(
                        xs_sh.at[row_v.at[ch + 2]], g[b2], gsem[b2])
            return 0
        lax.fori_loop(0, nch // 4, body, 0)

        for ch in (nch - 2, nch - 1):
            pltpu.make_async_copy(
                g[ch % 4], acc_sh.at[col_v.at[ch]], ssem[ch % 4]).wait()
        plsc.subcore_barrier()
        pltpu.sync_copy(
            acc_sh.at[pl.ds(sid * ROWS_PER_SUB, ROWS_PER_SUB)],
            acc_out.at[cid, pl.ds(sid * ROWS_PER_SUB, ROWS_PER_SUB)],
        )

    return agg_kernel


def _dis_from_deg(degt_blk):
    deg = degt_blk[:, 0:1] + degt_blk[:, 1:2] + 1.0
    return jnp.where(deg > 0, lax.rsqrt(jnp.maximum(deg, 1e-12)), 0.0)


def _mm1_body(x_ref, w_ref, degt_ref, xs_ref):
    dis = _dis_from_deg(degt_ref[...])
    xw = jnp.dot(x_ref[...], w_ref[...], preferred_element_type=jnp.float32)
    xw = xw * dis
    dh = xw.shape[1] // 2
    xs_ref[0] = xw[:, :dh]
    xs_ref[1] = xw[:, dh:]


def _mm2_body(accp_ref, xsp_ref, degt_ref, b1_ref, w2_ref, xs2_ref):
    dis = _dis_from_deg(degt_ref[...])
    acc = jnp.concatenate(
        [accp_ref[0] + xsp_ref[0], accp_ref[1] + xsp_ref[1]], axis=-1)
    h = jnp.maximum(acc * dis + b1_ref[...], 0.0)
    xw2 = jnp.dot(h, w2_ref[...], preferred_element_type=jnp.float32) * dis
    dh = xw2.shape[1] // 2
    xs2_ref[0] = xw2[:, :dh]
    xs2_ref[1] = xw2[:, dh:]


def _final_body(accp_ref, xsp_ref, degt_ref, b2_ref, out_ref):
    dis = _dis_from_deg(degt_ref[...])
    acc = jnp.concatenate(
        [accp_ref[0] + xsp_ref[0], accp_ref[1] + xsp_ref[1]], axis=-1)
    out_ref[...] = jnp.maximum(acc * dis + b2_ref[...], 0.0)


def kernel(x, edge_index, edge_weight, W1, b1, W2, b2):
    n, d_in = x.shape
    d = W1.shape[1]
    dh = d // 2
    e = edge_weight.shape[0]

    per_w = -(-e // NW)
    nch_d = -(-per_w // CH)
    e_pad_d = NW * nch_d * CH
    per_t = -(-e // NS)
    nch_a = -(-(-(-per_t // CH)) // 4) * 4
    e_pad_a = NS * nch_a * CH
    n_pad = NS * ROWS_PER_SUB

    row = edge_index[0].astype(jnp.int32)
    col = edge_index[1].astype(jnp.int32)
    ew = edge_weight.astype(jnp.float32)
    col_d = jnp.pad(col, (0, e_pad_d - e)).reshape(NW, nch_d, CH)
    ew_d = jnp.pad(ew, (0, e_pad_d - e)).reshape(NW, nch_d, CH)
    row_a = jnp.pad(row, (0, e_pad_a - e)).reshape(NS, nch_a, CH)
    col_a = jnp.pad(col, (0, e_pad_a - e)).reshape(NS, nch_a, CH)
    ew_a = jnp.pad(ew, (0, e_pad_a - e)).reshape(NS, nch_a * CH)

    deg_part = _make_deg_kernel(nch_d, n_pad)(col_d, ew_d)
    degt = jnp.transpose(deg_part)[:n]

    R = 1000
    grid = (n // R,)
    degt_spec = pl.BlockSpec((R, 2), lambda i: (i, 0))
    half_spec = pl.BlockSpec((2, R, dh), lambda i: (0, i, 0))
    b_spec = pl.BlockSpec((1, d), lambda i: (0, 0))
    half_shape = jax.ShapeDtypeStruct((2, n, dh), jnp.float32)

    xs1 = pl.pallas_call(
        _mm1_body,
        grid=grid,
        in_specs=[pl.BlockSpec((R, d_in), lambda i: (i, 0)),
                  pl.BlockSpec((d_in, d), lambda i: (0, 0)),
                  degt_spec],
        out_specs=half_spec,
        out_shape=half_shape,
    )(x, W1, degt)

    agg = _make_agg_kernel(nch_a, n_pad, dh)
    acc1 = agg(xs1, row_a, col_a, ew_a)

    xs2 = pl.pallas_call(
        _mm2_body,
        grid=grid,
        in_specs=[half_spec, half_spec, degt_spec, b_spec,
                  pl.BlockSpec((d, d), lambda i: (0, 0))],
        out_specs=half_spec,
        out_shape=half_shape,
    )(acc1, xs1, degt, b1.reshape(1, d), W2)

    acc2 = agg(xs2, row_a, col_a, ew_a)

    out = pl.pallas_call(
        _final_body,
        grid=grid,
        in_specs=[half_spec, half_spec, degt_spec, b_spec],
        out_specs=pl.BlockSpec((R, d), lambda i: (i, 0)),
        out_shape=jax.ShapeDtypeStruct((n, d), jnp.float32),
    )(acc2, xs2, degt, b2.reshape(1, d))

    return out
